# baseline (speedup 1.0000x reference)
import functools

import jax
import jax.numpy as jnp
from jax import lax
from jax.experimental import pallas as pl
from jax.experimental.pallas import tpu as pltpu
from jax.experimental.pallas import tpu_sc as plsc

_BATCH = 4
_SEQ = 8192
_D = 1024
_NC = 2
_NS = 16
_NW = _NC * _NS
_ROWS_PER_W = _SEQ // _NW
_CH = 32
_NCHUNK = _ROWS_PER_W // _CH


def _make():
    mesh = plsc.VectorSubcoreMesh(core_axis_name="c", subcore_axis_name="s")

    @functools.partial(
        pl.kernel,
        mesh=mesh,
        out_type=jax.ShapeDtypeStruct((_BATCH, _SEQ, _D), jnp.float32),
        scratch_types=[
            pltpu.VMEM((_CH, _D), jnp.float32),
            pltpu.VMEM((_CH, _D), jnp.float32),
            pltpu.SemaphoreType.DMA,
            pltpu.SemaphoreType.DMA,
            pltpu.SemaphoreType.DMA,
        ],
    )
    def k(table_hbm, out_hbm, buf0, buf1, rsem, wsem0, wsem1):
        wid = lax.axis_index("s") * _NC + lax.axis_index("c")
        base = wid * _ROWS_PER_W
        bufs = (buf0, buf1)
        wsems = (wsem0, wsem1)
        reads = [None] * _NCHUNK
        writes = [None] * _NCHUNK
        reads[0] = pltpu.async_copy(
            table_hbm.at[pl.ds(base, _CH)], bufs[0], rsem)
        for c in range(_NCHUNK):
            if c + 1 < _NCHUNK:
                if c >= 1:
                    for d in writes[c - 1]:
                        d.wait()
                reads[c + 1] = pltpu.async_copy(
                    table_hbm.at[pl.ds(base + (c + 1) * _CH, _CH)],
                    bufs[(c + 1) % 2], rsem)
            reads[c].wait()
            r0 = base + c * _CH
            writes[c] = [
                pltpu.async_copy(
                    bufs[c % 2], out_hbm.at[b].at[pl.ds(r0, _CH)], wsems[c % 2])
                for b in range(_BATCH)
            ]
        for c in (_NCHUNK - 2, _NCHUNK - 1):
            for d in writes[c]:
                d.wait()

    return k


_sc_kernel = _make()


def kernel(x, table):
    return _sc_kernel(table)


# speedup vs baseline: 1.0086x; 1.0086x over previous
import functools

import jax
import jax.numpy as jnp
from jax import lax
from jax.experimental import pallas as pl
from jax.experimental.pallas import tpu as pltpu
from jax.experimental.pallas import tpu_sc as plsc

_BATCH = 4
_SEQ = 8192
_D = 1024
_NC = 2
_NS = 16
_NW = _NC * _NS
_ROWS_PER_W = _SEQ // _NW
_CH = 32
_NBUF = 3
_NCHUNK = _ROWS_PER_W // _CH


def _make():
    mesh = plsc.VectorSubcoreMesh(core_axis_name="c", subcore_axis_name="s")

    @functools.partial(
        pl.kernel,
        mesh=mesh,
        out_type=jax.ShapeDtypeStruct((_BATCH, _SEQ, _D), jnp.float32),
        scratch_types=[
            pltpu.VMEM((_CH, _D), jnp.float32),
            pltpu.VMEM((_CH, _D), jnp.float32),
            pltpu.VMEM((_CH, _D), jnp.float32),
            pltpu.SemaphoreType.DMA,
            pltpu.SemaphoreType.DMA,
            pltpu.SemaphoreType.DMA,
            pltpu.SemaphoreType.DMA,
        ],
    )
    def k(table_hbm, out_hbm, buf0, buf1, buf2, rsem, wsem0, wsem1, wsem2):
        wid = lax.axis_index("s") * _NC + lax.axis_index("c")
        base = wid * _ROWS_PER_W
        bufs = (buf0, buf1, buf2)
        wsems = (wsem0, wsem1, wsem2)
        reads = [None] * _NCHUNK
        writes = [None] * _NCHUNK
        for c in range(min(_NBUF, _NCHUNK)):
            reads[c] = pltpu.async_copy(
                table_hbm.at[pl.ds(base + c * _CH, _CH)], bufs[c % _NBUF], rsem)
        for c in range(_NCHUNK):
            reads[c].wait()
            r0 = base + c * _CH
            writes[c] = [
                pltpu.async_copy(
                    bufs[c % _NBUF], out_hbm.at[b].at[pl.ds(r0, _CH)],
                    wsems[c % _NBUF])
                for b in range(_BATCH)
            ]
            nxt = c + _NBUF
            if nxt < _NCHUNK:
                for d in writes[nxt - _NBUF]:
                    d.wait()
                reads[nxt] = pltpu.async_copy(
                    table_hbm.at[pl.ds(base + nxt * _CH, _CH)],
                    bufs[nxt % _NBUF], rsem)
        for c in range(max(0, _NCHUNK - _NBUF), _NCHUNK):
            for d in writes[c]:
                d.wait()

    return k


_sc_kernel = _make()


def kernel(x, table):
    return _sc_kernel(table)


# speedup vs baseline: 1.0121x; 1.0035x over previous
import functools

import jax
import jax.numpy as jnp
from jax import lax
from jax.experimental import pallas as pl
from jax.experimental.pallas import tpu as pltpu
from jax.experimental.pallas import tpu_sc as plsc

_NC = 2
_NS = 16
_NW = _NC * _NS
_CH = 32
_NBUF = 3


@functools.lru_cache(maxsize=None)
def _make(batch, seq_len, d_model, dtype):
    rows_per_w = seq_len // _NW
    n_chunks = rows_per_w // _CH
    mesh = plsc.VectorSubcoreMesh(core_axis_name="c", subcore_axis_name="s")

    @functools.partial(
        pl.kernel,
        mesh=mesh,
        out_type=jax.ShapeDtypeStruct((batch, seq_len, d_model), dtype),
        scratch_types=[
            pltpu.VMEM((_CH, d_model), dtype),
            pltpu.VMEM((_CH, d_model), dtype),
            pltpu.VMEM((_CH, d_model), dtype),
            pltpu.SemaphoreType.DMA,
            pltpu.SemaphoreType.DMA,
            pltpu.SemaphoreType.DMA,
            pltpu.SemaphoreType.DMA,
        ],
    )
    def k(table_hbm, out_hbm, buf0, buf1, buf2, rsem, wsem0, wsem1, wsem2):
        wid = lax.axis_index("s") * _NC + lax.axis_index("c")
        base = wid * rows_per_w
        bufs = (buf0, buf1, buf2)
        wsems = (wsem0, wsem1, wsem2)
        reads = [None] * n_chunks
        writes = [None] * n_chunks
        for c in range(min(_NBUF, n_chunks)):
            reads[c] = pltpu.async_copy(
                table_hbm.at[pl.ds(base + c * _CH, _CH)], bufs[c % _NBUF], rsem)
        for c in range(n_chunks):
            reads[c].wait()
            r0 = base + c * _CH
            writes[c] = [
                pltpu.async_copy(
                    bufs[c % _NBUF], out_hbm.at[b].at[pl.ds(r0, _CH)],
                    wsems[c % _NBUF])
                for b in range(batch)
            ]
            nxt = c + _NBUF
            if nxt < n_chunks:
                for d in writes[c]:
                    d.wait()
                reads[nxt] = pltpu.async_copy(
                    table_hbm.at[pl.ds(base + nxt * _CH, _CH)],
                    bufs[nxt % _NBUF], rsem)
        for c in range(max(0, n_chunks - _NBUF), n_chunks):
            for d in writes[c]:
                d.wait()

    return k


def kernel(x, table):
    batch, seq_len = x.shape
    d_model = table.shape[1]
    return _make(batch, seq_len, d_model, table.dtype)(table)
